# Initial kernel scaffold; baseline (speedup 1.0000x reference)
import jax
import jax.numpy as jnp
from jax.experimental import pallas as pl


def kernel(hidden_states, attention_mask, pos_cos, pos_sin, input_ln_w, post_ln_w, qkv_w, o_w, q_ln_w, k_ln_w, gate_w, shared_w_gu, shared_w_down, exp_w_gu, exp_w_down):
    raise NotImplementedError("write your pallas kernel here")


# speedup vs baseline: 1.7459x; 1.7459x over previous
import functools
import math

import jax
import jax.numpy as jnp
from jax import lax
from jax.experimental import pallas as pl
from jax.experimental.pallas import tpu as pltpu
from jax.experimental.pallas import tpu_sc as plsc

B, S, H = 1, 2048, 768
NH, NKV, HD = 16, 8, 48
E, K, FF = 64, 8, 6144
HHD = HD // 2
G = NH // NKV

TILE = 128
NT = S * K // TILE + E
PAD = NT * TILE

NW = 32
_SC_MESH = dict(core_axis_name="c", subcore_axis_name="s")


def _prologue_body(hid_ref, iw_ref, wq1_ref, wq2_ref, wk1_ref, wk2_ref,
                   wv_ref, cosq_ref, sinq_ref, cosk_ref, sink_ref,
                   qlw1_ref, qlw2_ref, klw1_ref, klw2_ref,
                   sq_ref, sqt_ref, sk_ref, skt_ref,
                   q1_ref, q2_ref, k1_ref, k2_ref, v_ref):
    x = hid_ref[...]
    var = jnp.mean(x * x, axis=1, keepdims=True)
    h = iw_ref[...] * (x * lax.rsqrt(var + 1e-6))
    dot = functools.partial(jnp.dot, preferred_element_type=jnp.float32)
    q1 = dot(h, wq1_ref[...])
    q2 = dot(h, wq2_ref[...])
    k1 = dot(h, wk1_ref[...])
    k2 = dot(h, wk2_ref[...])
    v_ref[...] = dot(h, wv_ref[...])
    cq, sq_ = cosq_ref[...], sinq_ref[...]
    ck, sk_ = cosk_ref[...], sink_ref[...]
    q1r = q1 * cq - q2 * sq_
    q2r = q2 * cq + q1 * sq_
    k1r = k1 * ck - k2 * sk_
    k2r = k2 * ck + k1 * sk_
    ssq = dot(q1r * q1r + q2r * q2r, sq_ref[...])
    scq = lax.rsqrt(ssq / HD + 1e-6)
    sbq = dot(scq, sqt_ref[...])
    ssk = dot(k1r * k1r + k2r * k2r, sk_ref[...])
    sck = lax.rsqrt(ssk / HD + 1e-6)
    sbk = dot(sck, skt_ref[...])
    q1_ref[...] = q1r * sbq * qlw1_ref[...]
    q2_ref[...] = q2r * sbq * qlw2_ref[...]
    k1_ref[...] = k1r * sbk * klw1_ref[...]
    k2_ref[...] = k2r * sbk * klw2_ref[...]


def _attn_body(q_ref, k_ref, v_ref, mask_ref, o_ref):
    q = q_ref[0]
    k = k_ref[0]
    v = v_ref[0]
    s = lax.dot_general(q, k, (((1,), (1,)), ((), ())),
                        preferred_element_type=jnp.float32)
    s = s * (1.0 / math.sqrt(HD)) + mask_ref[...]
    m = jnp.max(s, axis=1, keepdims=True)
    p = jnp.exp(s - m)
    l = jnp.sum(p, axis=1, keepdims=True)
    o = jnp.dot(p, v, preferred_element_type=jnp.float32)
    o_ref[0] = o / l


def _post_body(o2d_ref, hid_ref, ow_ref, pw_ref, gw_ref, wgu_ref, wdn_ref,
               base_ref, h2_ref, logits_ref):
    dot = functools.partial(jnp.dot, preferred_element_type=jnp.float32)
    att = dot(o2d_ref[...], ow_ref[...])
    hs2 = hid_ref[...] + att
    var = jnp.mean(hs2 * hs2, axis=1, keepdims=True)
    h2 = pw_ref[...] * (hs2 * lax.rsqrt(var + 1e-6))
    h2_ref[...] = h2
    logits_ref[...] = dot(h2, gw_ref[...])
    hm = dot(h2, wgu_ref[...])
    x1 = hm[:, :FF // 2]
    x2 = hm[:, FF // 2:]
    act = x1 * (x2 * jax.nn.sigmoid(x2))
    base_ref[...] = hs2 + dot(act, wdn_ref[...])


def _expert_body(eids_ref, valid_ref, x_ref, w_ref, wgu_ref, wdn_ref, out_ref):
    i = pl.program_id(0)
    out_ref[...] = jnp.zeros_like(out_ref)

    @pl.when(valid_ref[i] == 1)
    def _():
        x = x_ref[...]
        hm = jnp.dot(x, wgu_ref[0], preferred_element_type=jnp.float32)
        x1 = hm[:, :FF // 2]
        x2 = hm[:, FF // 2:]
        act = x1 * (x2 * jax.nn.sigmoid(x2))
        o = jnp.dot(act, wdn_ref[0], preferred_element_type=jnp.float32)
        out_ref[...] = o * w_ref[...]


def _dispatch_body(h2_hbm, tok_hbm, xs_hbm, idx_v, rows_v, sem):
    wid = lax.axis_index("s") * 2 + lax.axis_index("c")
    rows_per_w = PAD // NW
    base = wid * rows_per_w
    for c in range(rows_per_w // TILE):
        st = base + c * TILE
        pltpu.sync_copy(tok_hbm.at[pl.ds(st, TILE)], idx_v)
        pltpu.async_copy(h2_hbm.at[idx_v], rows_v, sem).wait()
        pltpu.sync_copy(rows_v, xs_hbm.at[pl.ds(st, TILE)])


def _dispatch(h2, tok_pad):
    mesh = plsc.VectorSubcoreMesh(**_SC_MESH)
    f = pl.kernel(
        _dispatch_body,
        out_type=jax.ShapeDtypeStruct((PAD, H), jnp.float32),
        mesh=mesh,
        scratch_types=[
            pltpu.VMEM((TILE,), jnp.int32),
            pltpu.VMEM((TILE, H), jnp.float32),
            pltpu.SemaphoreType.DMA,
        ],
    )
    return f(h2, tok_pad)


_CTOK = 16


def _combine_body(eo_hbm, psa_hbm, base_hbm, out_hbm, idx_v, rows_v, acc_v, sem):
    wid = lax.axis_index("s") * 2 + lax.axis_index("c")
    tok_per_w = S // NW
    tbase = wid * tok_per_w
    for c in range(tok_per_w // _CTOK):
        t0 = tbase + c * _CTOK
        pltpu.sync_copy(psa_hbm.at[pl.ds(t0 * K, _CTOK * K)], idx_v)
        pltpu.async_copy(eo_hbm.at[idx_v], rows_v, sem).wait()
        pltpu.sync_copy(base_hbm.at[pl.ds(t0, _CTOK)], acc_v)

        def jbody(j, _):
            for t in range(_CTOK):
                a = acc_v[t, pl.ds(j * 16, 16)]
                for k in range(K):
                    a = a + rows_v[t * K + k, pl.ds(j * 16, 16)]
                acc_v[t, pl.ds(j * 16, 16)] = a
            return 0

        lax.fori_loop(0, H // 16, jbody, 0)
        pltpu.sync_copy(acc_v, out_hbm.at[pl.ds(t0, _CTOK)])


def _combine(eo, ps_of_a, base):
    mesh = plsc.VectorSubcoreMesh(**_SC_MESH)
    f = pl.kernel(
        _combine_body,
        out_type=jax.ShapeDtypeStruct((S, H), jnp.float32),
        mesh=mesh,
        scratch_types=[
            pltpu.VMEM((_CTOK * K,), jnp.int32),
            pltpu.VMEM((_CTOK * K, H), jnp.float32),
            pltpu.VMEM((_CTOK, H), jnp.float32),
            pltpu.SemaphoreType.DMA,
        ],
    )
    return f(eo, ps_of_a, base)


def kernel(hidden_states, attention_mask, pos_cos, pos_sin, input_ln_w,
           post_ln_w, qkv_w, o_w, q_ln_w, k_ln_w, gate_w, shared_w_gu,
           shared_w_down, exp_w_gu, exp_w_down):
    f32 = jnp.float32
    x0 = hidden_states.reshape(S, H)

    wr = qkv_w.reshape(H, NKV, G + 2, HD)
    wq = wr[:, :, :G, :].reshape(H, NH, HD)
    wk = wr[:, :, G, :].reshape(H, NKV, HD)
    wv = wr[:, :, G + 1, :].reshape(H, NKV * HD)
    wq1 = wq[:, :, :HHD].reshape(H, NH * HHD)
    wq2 = wq[:, :, HHD:].reshape(H, NH * HHD)
    wk1 = wk[:, :, :HHD].reshape(H, NKV * HHD)
    wk2 = wk[:, :, HHD:].reshape(H, NKV * HHD)
    cos_h = pos_cos[0, :, :HHD]
    sin_h = pos_sin[0, :, :HHD]
    cosq = jnp.tile(cos_h, (1, NH))
    sinq = jnp.tile(sin_h, (1, NH))
    cosk = jnp.tile(cos_h, (1, NKV))
    sink = jnp.tile(sin_h, (1, NKV))
    qlw1 = jnp.tile(q_ln_w[:HHD], NH)[None, :]
    qlw2 = jnp.tile(q_ln_w[HHD:], NH)[None, :]
    klw1 = jnp.tile(k_ln_w[:HHD], NKV)[None, :]
    klw2 = jnp.tile(k_ln_w[HHD:], NKV)[None, :]
    eyeq = jnp.eye(NH, dtype=f32)
    sq_m = jnp.repeat(eyeq, HHD, axis=0)
    sqt_m = jnp.repeat(eyeq, HHD, axis=1)
    eyek = jnp.eye(NKV, dtype=f32)
    sk_m = jnp.repeat(eyek, HHD, axis=0)
    skt_m = jnp.repeat(eyek, HHD, axis=1)

    prologue = pl.pallas_call(
        _prologue_body,
        out_shape=[
            jax.ShapeDtypeStruct((S, NH * HHD), f32),
            jax.ShapeDtypeStruct((S, NH * HHD), f32),
            jax.ShapeDtypeStruct((S, NKV * HHD), f32),
            jax.ShapeDtypeStruct((S, NKV * HHD), f32),
            jax.ShapeDtypeStruct((S, NKV * HD), f32),
        ],
        compiler_params=pltpu.CompilerParams(vmem_limit_bytes=120 * 2**20),
    )
    q1, q2, k1, k2, v2d = prologue(
        x0, input_ln_w[None, :], wq1, wq2, wk1, wk2, wv, cosq, sinq, cosk,
        sink, qlw1, qlw2, klw1, klw2, sq_m, sqt_m, sk_m, skt_m)

    q = jnp.concatenate([q1.reshape(S, NH, HHD), q2.reshape(S, NH, HHD)],
                        axis=-1).transpose(1, 0, 2)
    k = jnp.concatenate([k1.reshape(S, NKV, HHD), k2.reshape(S, NKV, HHD)],
                        axis=-1).transpose(1, 0, 2)
    k = jnp.repeat(k, G, axis=0)
    v = v2d.reshape(S, NKV, HD).transpose(1, 0, 2)
    v = jnp.repeat(v, G, axis=0)
    mask2d = attention_mask.reshape(S, S)

    attn = pl.pallas_call(
        _attn_body,
        grid=(NH,),
        in_specs=[
            pl.BlockSpec((1, S, HD), lambda h: (h, 0, 0)),
            pl.BlockSpec((1, S, HD), lambda h: (h, 0, 0)),
            pl.BlockSpec((1, S, HD), lambda h: (h, 0, 0)),
            pl.BlockSpec((S, S), lambda h: (0, 0)),
        ],
        out_specs=pl.BlockSpec((1, S, HD), lambda h: (h, 0, 0)),
        out_shape=jax.ShapeDtypeStruct((NH, S, HD), f32),
        compiler_params=pltpu.CompilerParams(vmem_limit_bytes=120 * 2**20),
    )
    o3 = attn(q, k, v, mask2d)
    o2d = o3.transpose(1, 0, 2).reshape(S, NH * HD)

    RB = 512
    post = pl.pallas_call(
        _post_body,
        grid=(S // RB,),
        in_specs=[
            pl.BlockSpec((RB, H), lambda i: (i, 0)),
            pl.BlockSpec((RB, H), lambda i: (i, 0)),
            pl.BlockSpec((H, H), lambda i: (0, 0)),
            pl.BlockSpec((1, H), lambda i: (0, 0)),
            pl.BlockSpec((H, E), lambda i: (0, 0)),
            pl.BlockSpec((H, FF), lambda i: (0, 0)),
            pl.BlockSpec((FF // 2, H), lambda i: (0, 0)),
        ],
        out_specs=[
            pl.BlockSpec((RB, H), lambda i: (i, 0)),
            pl.BlockSpec((RB, H), lambda i: (i, 0)),
            pl.BlockSpec((RB, E), lambda i: (i, 0)),
        ],
        out_shape=[
            jax.ShapeDtypeStruct((S, H), f32),
            jax.ShapeDtypeStruct((S, H), f32),
            jax.ShapeDtypeStruct((S, E), f32),
        ],
        compiler_params=pltpu.CompilerParams(vmem_limit_bytes=120 * 2**20),
    )
    base, h2, logits = post(o2d, x0, o_w, post_ln_w[None, :], gate_w,
                            shared_w_gu, shared_w_down)

    i32 = jnp.int32
    gates = jax.nn.softmax(logits, axis=1)
    topv, topi = lax.top_k(gates, K)
    gsum = jnp.maximum(jnp.sum(topv, axis=1, keepdims=True), 1.1920929e-07)
    wts = topv / gsum
    eid = topi.reshape(-1).astype(i32)
    order = jnp.argsort(eid, stable=True).astype(i32)
    counts = jnp.bincount(eid, length=E).astype(i32)
    offs = jnp.concatenate([jnp.zeros((1,), i32),
                            jnp.cumsum(counts)[:-1].astype(i32)])
    nt = (counts + TILE - 1) // TILE
    tcum = jnp.cumsum(nt).astype(i32)
    po = jnp.concatenate([jnp.zeros((1,), i32),
                          tcum[:-1]]) * TILE
    tile_ids = jnp.arange(NT, dtype=i32)
    tile_eid = jnp.searchsorted(tcum, tile_ids, side="right").astype(i32)
    tile_valid = (tile_ids < tcum[-1]).astype(i32)
    tile_eid = jnp.minimum(tile_eid, E - 1)
    ps = jnp.arange(PAD, dtype=i32)
    e_ps = tile_eid[ps // TILE]
    inner = ps - po[e_ps]
    slot_valid = inner < counts[e_ps]
    src = offs[e_ps] + jnp.where(slot_valid, inner, 0)
    a_ps = order[src]
    tok_pad = jnp.where(slot_valid, a_ps // K, 0).astype(i32)
    w_pad = jnp.where(slot_valid, wts.reshape(-1)[a_ps], 0.0)[:, None]
    eid_sorted = eid[order]
    s_idx = jnp.arange(S * K, dtype=i32)
    ps_sorted = po[eid_sorted] + (s_idx - offs[eid_sorted])
    ps_of_a = jnp.zeros((S * K,), i32).at[order].set(ps_sorted)

    xs = _dispatch(h2, tok_pad)

    grid_spec = pltpu.PrefetchScalarGridSpec(
        num_scalar_prefetch=2,
        grid=(NT,),
        in_specs=[
            pl.BlockSpec((TILE, H), lambda i, eids, vs: (i, 0)),
            pl.BlockSpec((TILE, 1), lambda i, eids, vs: (i, 0)),
            pl.BlockSpec((1, H, FF), lambda i, eids, vs: (eids[i], 0, 0)),
            pl.BlockSpec((1, FF // 2, H), lambda i, eids, vs: (eids[i], 0, 0)),
        ],
        out_specs=pl.BlockSpec((TILE, H), lambda i, eids, vs: (i, 0)),
    )
    expert_ffn = pl.pallas_call(
        _expert_body,
        grid_spec=grid_spec,
        out_shape=jax.ShapeDtypeStruct((PAD, H), f32),
        compiler_params=pltpu.CompilerParams(vmem_limit_bytes=120 * 2**20),
    )
    eo = expert_ffn(tile_eid, tile_valid, xs, w_pad, exp_w_gu, exp_w_down)

    out = _combine(eo, ps_of_a, base)
    return out.reshape(B, S, H)


# speedup vs baseline: 1.7680x; 1.0127x over previous
import functools
import math

import jax
import jax.numpy as jnp
from jax import lax
from jax.experimental import pallas as pl
from jax.experimental.pallas import tpu as pltpu
from jax.experimental.pallas import tpu_sc as plsc

B, S, H = 1, 2048, 768
NH, NKV, HD = 16, 8, 48
E, K, FF = 64, 8, 6144
HHD = HD // 2
G = NH // NKV

TILE = 128
NT = S * K // TILE + E
PAD = NT * TILE

NW = 32
_SC_MESH = dict(core_axis_name="c", subcore_axis_name="s")


def _prologue_body(hid_ref, iw_ref, wq1_ref, wq2_ref, wk1_ref, wk2_ref,
                   wv_ref, cosq_ref, sinq_ref, cosk_ref, sink_ref,
                   qlw1_ref, qlw2_ref, klw1_ref, klw2_ref,
                   sq_ref, sqt_ref, sk_ref, skt_ref,
                   q1_ref, q2_ref, k1_ref, k2_ref, v_ref):
    x = hid_ref[...]
    var = jnp.mean(x * x, axis=1, keepdims=True)
    h = iw_ref[...] * (x * lax.rsqrt(var + 1e-6))
    dot = functools.partial(jnp.dot, preferred_element_type=jnp.float32)
    q1 = dot(h, wq1_ref[...])
    q2 = dot(h, wq2_ref[...])
    k1 = dot(h, wk1_ref[...])
    k2 = dot(h, wk2_ref[...])
    v_ref[...] = dot(h, wv_ref[...])
    cq, sq_ = cosq_ref[...], sinq_ref[...]
    ck, sk_ = cosk_ref[...], sink_ref[...]
    q1r = q1 * cq - q2 * sq_
    q2r = q2 * cq + q1 * sq_
    k1r = k1 * ck - k2 * sk_
    k2r = k2 * ck + k1 * sk_
    ssq = dot(q1r * q1r + q2r * q2r, sq_ref[...])
    scq = lax.rsqrt(ssq / HD + 1e-6)
    sbq = dot(scq, sqt_ref[...])
    ssk = dot(k1r * k1r + k2r * k2r, sk_ref[...])
    sck = lax.rsqrt(ssk / HD + 1e-6)
    sbk = dot(sck, skt_ref[...])
    q1_ref[...] = q1r * sbq * qlw1_ref[...]
    q2_ref[...] = q2r * sbq * qlw2_ref[...]
    k1_ref[...] = k1r * sbk * klw1_ref[...]
    k2_ref[...] = k2r * sbk * klw2_ref[...]


def _attn_body(q_ref, k_ref, v_ref, mask_ref, o_ref):
    q = q_ref[0]
    k = k_ref[0]
    v = v_ref[0]
    s = lax.dot_general(q, k, (((1,), (1,)), ((), ())),
                        preferred_element_type=jnp.float32)
    s = s * (1.0 / math.sqrt(HD)) + mask_ref[...]
    m = jnp.max(s, axis=1, keepdims=True)
    p = jnp.exp(s - m)
    l = jnp.sum(p, axis=1, keepdims=True)
    o = jnp.dot(p, v, preferred_element_type=jnp.float32)
    o_ref[0] = o / l


def _post_body(o2d_ref, hid_ref, ow_ref, pw_ref, gw_ref,
               hs2_ref, h2_ref, logits_ref):
    dot = functools.partial(jnp.dot, preferred_element_type=jnp.float32)
    att = dot(o2d_ref[...], ow_ref[...])
    hs2 = hid_ref[...] + att
    var = jnp.mean(hs2 * hs2, axis=1, keepdims=True)
    h2 = pw_ref[...] * (hs2 * lax.rsqrt(var + 1e-6))
    hs2_ref[...] = hs2
    h2_ref[...] = h2
    logits_ref[...] = dot(h2, gw_ref[...])


def _shared_body(hs2_ref, h2_ref, wgu_ref, wdn_ref, base_ref):
    dot = functools.partial(jnp.dot, preferred_element_type=jnp.float32)
    hm = dot(h2_ref[...], wgu_ref[...])
    x1 = hm[:, :FF // 2]
    x2 = hm[:, FF // 2:]
    act = x1 * (x2 * jax.nn.sigmoid(x2))
    base_ref[...] = hs2_ref[...] + dot(act, wdn_ref[...])


def _expert_body(eids_ref, valid_ref, x_ref, w_ref, wgu_ref, wdn_ref, out_ref):
    i = pl.program_id(0)
    out_ref[...] = jnp.zeros_like(out_ref)

    @pl.when(valid_ref[i] == 1)
    def _():
        x = x_ref[...]
        hm = jnp.dot(x, wgu_ref[0], preferred_element_type=jnp.float32)
        x1 = hm[:, :FF // 2]
        x2 = hm[:, FF // 2:]
        act = x1 * (x2 * jax.nn.sigmoid(x2))
        o = jnp.dot(act, wdn_ref[0], preferred_element_type=jnp.float32)
        out_ref[...] = o * w_ref[...]


DCH = 64


def _dispatch_body(h2_hbm, tok_hbm, xs_hbm, idx_v, rows0, rows1,
                   gsem0, gsem1, osem0, osem1):
    wid = lax.axis_index("s") * 2 + lax.axis_index("c")
    rows_per_w = PAD // NW
    base = wid * rows_per_w
    nch = rows_per_w // DCH
    pltpu.sync_copy(tok_hbm.at[pl.ds(base, rows_per_w)], idx_v)
    rows = [rows0, rows1]
    gsems = [gsem0, gsem1]
    osems = [osem0, osem1]
    gh = [None, None]
    wh = [None, None]
    gh[0] = pltpu.async_copy(h2_hbm.at[idx_v.at[pl.ds(0, DCH)]],
                             rows[0], gsems[0])
    for c in range(nch):
        b = c & 1
        nb = 1 - b
        if c + 1 < nch:
            if wh[nb] is not None:
                wh[nb].wait()
            gh[nb] = pltpu.async_copy(
                h2_hbm.at[idx_v.at[pl.ds((c + 1) * DCH, DCH)]],
                rows[nb], gsems[nb])
        gh[b].wait()
        wh[b] = pltpu.async_copy(rows[b],
                                 xs_hbm.at[pl.ds(base + c * DCH, DCH)],
                                 osems[b])
    for h in wh:
        if h is not None:
            h.wait()


def _dispatch(h2, tok_pad):
    mesh = plsc.VectorSubcoreMesh(**_SC_MESH)
    f = pl.kernel(
        _dispatch_body,
        out_type=jax.ShapeDtypeStruct((PAD, H), jnp.float32),
        mesh=mesh,
        scratch_types=[
            pltpu.VMEM((PAD // NW,), jnp.int32),
            pltpu.VMEM((DCH, H), jnp.float32),
            pltpu.VMEM((DCH, H), jnp.float32),
            pltpu.SemaphoreType.DMA,
            pltpu.SemaphoreType.DMA,
            pltpu.SemaphoreType.DMA,
            pltpu.SemaphoreType.DMA,
        ],
    )
    return f(h2, tok_pad)


_CTOK = 16


def _combine_body(eo_hbm, psa_hbm, base_hbm, out_hbm, idx_v, rows_v, acc_v, sem):
    wid = lax.axis_index("s") * 2 + lax.axis_index("c")
    tok_per_w = S // NW
    tbase = wid * tok_per_w
    for c in range(tok_per_w // _CTOK):
        t0 = tbase + c * _CTOK
        pltpu.sync_copy(psa_hbm.at[pl.ds(t0 * K, _CTOK * K)], idx_v)
        pltpu.async_copy(eo_hbm.at[idx_v], rows_v, sem).wait()
        pltpu.sync_copy(base_hbm.at[pl.ds(t0, _CTOK)], acc_v)

        def jbody(j, _):
            for t in range(_CTOK):
                a = acc_v[t, pl.ds(j * 16, 16)]
                for k in range(K):
                    a = a + rows_v[t * K + k, pl.ds(j * 16, 16)]
                acc_v[t, pl.ds(j * 16, 16)] = a
            return 0

        lax.fori_loop(0, H // 16, jbody, 0)
        pltpu.sync_copy(acc_v, out_hbm.at[pl.ds(t0, _CTOK)])


def _combine(eo, ps_of_a, base):
    mesh = plsc.VectorSubcoreMesh(**_SC_MESH)
    f = pl.kernel(
        _combine_body,
        out_type=jax.ShapeDtypeStruct((S, H), jnp.float32),
        mesh=mesh,
        scratch_types=[
            pltpu.VMEM((_CTOK * K,), jnp.int32),
            pltpu.VMEM((_CTOK * K, H), jnp.float32),
            pltpu.VMEM((_CTOK, H), jnp.float32),
            pltpu.SemaphoreType.DMA,
        ],
    )
    return f(eo, ps_of_a, base)


def kernel(hidden_states, attention_mask, pos_cos, pos_sin, input_ln_w,
           post_ln_w, qkv_w, o_w, q_ln_w, k_ln_w, gate_w, shared_w_gu,
           shared_w_down, exp_w_gu, exp_w_down):
    f32 = jnp.float32
    x0 = hidden_states.reshape(S, H)

    wr = qkv_w.reshape(H, NKV, G + 2, HD)
    wq = wr[:, :, :G, :].reshape(H, NH, HD)
    wk = wr[:, :, G, :].reshape(H, NKV, HD)
    wv = wr[:, :, G + 1, :].reshape(H, NKV * HD)
    wq1 = wq[:, :, :HHD].reshape(H, NH * HHD)
    wq2 = wq[:, :, HHD:].reshape(H, NH * HHD)
    wk1 = wk[:, :, :HHD].reshape(H, NKV * HHD)
    wk2 = wk[:, :, HHD:].reshape(H, NKV * HHD)
    cos_h = pos_cos[0, :, :HHD]
    sin_h = pos_sin[0, :, :HHD]
    cosq = jnp.tile(cos_h, (1, NH))
    sinq = jnp.tile(sin_h, (1, NH))
    cosk = jnp.tile(cos_h, (1, NKV))
    sink = jnp.tile(sin_h, (1, NKV))
    qlw1 = jnp.tile(q_ln_w[:HHD], NH)[None, :]
    qlw2 = jnp.tile(q_ln_w[HHD:], NH)[None, :]
    klw1 = jnp.tile(k_ln_w[:HHD], NKV)[None, :]
    klw2 = jnp.tile(k_ln_w[HHD:], NKV)[None, :]
    eyeq = jnp.eye(NH, dtype=f32)
    sq_m = jnp.repeat(eyeq, HHD, axis=0)
    sqt_m = jnp.repeat(eyeq, HHD, axis=1)
    eyek = jnp.eye(NKV, dtype=f32)
    sk_m = jnp.repeat(eyek, HHD, axis=0)
    skt_m = jnp.repeat(eyek, HHD, axis=1)

    prologue = pl.pallas_call(
        _prologue_body,
        out_shape=[
            jax.ShapeDtypeStruct((S, NH * HHD), f32),
            jax.ShapeDtypeStruct((S, NH * HHD), f32),
            jax.ShapeDtypeStruct((S, NKV * HHD), f32),
            jax.ShapeDtypeStruct((S, NKV * HHD), f32),
            jax.ShapeDtypeStruct((S, NKV * HD), f32),
        ],
        compiler_params=pltpu.CompilerParams(vmem_limit_bytes=120 * 2**20),
    )
    q1, q2, k1, k2, v2d = prologue(
        x0, input_ln_w[None, :], wq1, wq2, wk1, wk2, wv, cosq, sinq, cosk,
        sink, qlw1, qlw2, klw1, klw2, sq_m, sqt_m, sk_m, skt_m)

    q = jnp.concatenate([q1.reshape(S, NH, HHD), q2.reshape(S, NH, HHD)],
                        axis=-1).transpose(1, 0, 2)
    k = jnp.concatenate([k1.reshape(S, NKV, HHD), k2.reshape(S, NKV, HHD)],
                        axis=-1).transpose(1, 0, 2)
    k = jnp.repeat(k, G, axis=0)
    v = v2d.reshape(S, NKV, HD).transpose(1, 0, 2)
    v = jnp.repeat(v, G, axis=0)
    mask2d = attention_mask.reshape(S, S)

    attn = pl.pallas_call(
        _attn_body,
        grid=(NH,),
        in_specs=[
            pl.BlockSpec((1, S, HD), lambda h: (h, 0, 0)),
            pl.BlockSpec((1, S, HD), lambda h: (h, 0, 0)),
            pl.BlockSpec((1, S, HD), lambda h: (h, 0, 0)),
            pl.BlockSpec((S, S), lambda h: (0, 0)),
        ],
        out_specs=pl.BlockSpec((1, S, HD), lambda h: (h, 0, 0)),
        out_shape=jax.ShapeDtypeStruct((NH, S, HD), f32),
        compiler_params=pltpu.CompilerParams(vmem_limit_bytes=120 * 2**20),
    )
    o3 = attn(q, k, v, mask2d)
    o2d = o3.transpose(1, 0, 2).reshape(S, NH * HD)

    RB = 512
    post = pl.pallas_call(
        _post_body,
        grid=(S // RB,),
        in_specs=[
            pl.BlockSpec((RB, H), lambda i: (i, 0)),
            pl.BlockSpec((RB, H), lambda i: (i, 0)),
            pl.BlockSpec((H, H), lambda i: (0, 0)),
            pl.BlockSpec((1, H), lambda i: (0, 0)),
            pl.BlockSpec((H, E), lambda i: (0, 0)),
        ],
        out_specs=[
            pl.BlockSpec((RB, H), lambda i: (i, 0)),
            pl.BlockSpec((RB, H), lambda i: (i, 0)),
            pl.BlockSpec((RB, E), lambda i: (i, 0)),
        ],
        out_shape=[
            jax.ShapeDtypeStruct((S, H), f32),
            jax.ShapeDtypeStruct((S, H), f32),
            jax.ShapeDtypeStruct((S, E), f32),
        ],
        compiler_params=pltpu.CompilerParams(vmem_limit_bytes=120 * 2**20),
    )
    hs2, h2, logits = post(o2d, x0, o_w, post_ln_w[None, :], gate_w)

    shared = pl.pallas_call(
        _shared_body,
        grid=(S // RB,),
        in_specs=[
            pl.BlockSpec((RB, H), lambda i: (i, 0)),
            pl.BlockSpec((RB, H), lambda i: (i, 0)),
            pl.BlockSpec((H, FF), lambda i: (0, 0)),
            pl.BlockSpec((FF // 2, H), lambda i: (0, 0)),
        ],
        out_specs=pl.BlockSpec((RB, H), lambda i: (i, 0)),
        out_shape=jax.ShapeDtypeStruct((S, H), f32),
        compiler_params=pltpu.CompilerParams(vmem_limit_bytes=120 * 2**20),
    )
    base = shared(hs2, h2, shared_w_gu, shared_w_down)

    i32 = jnp.int32
    gates = jax.nn.softmax(logits, axis=1)
    topv, topi = lax.top_k(gates, K)
    gsum = jnp.maximum(jnp.sum(topv, axis=1, keepdims=True), 1.1920929e-07)
    wts = topv / gsum
    eid = topi.reshape(-1).astype(i32)
    order = jnp.argsort(eid, stable=True).astype(i32)
    counts = jnp.bincount(eid, length=E).astype(i32)
    offs = jnp.concatenate([jnp.zeros((1,), i32),
                            jnp.cumsum(counts)[:-1].astype(i32)])
    nt = (counts + TILE - 1) // TILE
    tcum = jnp.cumsum(nt).astype(i32)
    po = jnp.concatenate([jnp.zeros((1,), i32),
                          tcum[:-1]]) * TILE
    tile_ids = jnp.arange(NT, dtype=i32)
    tile_eid = jnp.searchsorted(tcum, tile_ids, side="right").astype(i32)
    tile_valid = (tile_ids < tcum[-1]).astype(i32)
    tile_eid = jnp.minimum(tile_eid, E - 1)
    ps = jnp.arange(PAD, dtype=i32)
    e_ps = tile_eid[ps // TILE]
    inner = ps - po[e_ps]
    slot_valid = inner < counts[e_ps]
    src = offs[e_ps] + jnp.where(slot_valid, inner, 0)
    a_ps = order[src]
    tok_pad = jnp.where(slot_valid, a_ps // K, 0).astype(i32)
    w_pad = jnp.where(slot_valid, wts.reshape(-1)[a_ps], 0.0)[:, None]
    eid_sorted = eid[order]
    s_idx = jnp.arange(S * K, dtype=i32)
    ps_sorted = po[eid_sorted] + (s_idx - offs[eid_sorted])
    ps_of_a = jnp.zeros((S * K,), i32).at[order].set(ps_sorted)

    xs = _dispatch(h2, tok_pad)

    grid_spec = pltpu.PrefetchScalarGridSpec(
        num_scalar_prefetch=2,
        grid=(NT,),
        in_specs=[
            pl.BlockSpec((TILE, H), lambda i, eids, vs: (i, 0)),
            pl.BlockSpec((TILE, 1), lambda i, eids, vs: (i, 0)),
            pl.BlockSpec((1, H, FF), lambda i, eids, vs: (eids[i], 0, 0)),
            pl.BlockSpec((1, FF // 2, H), lambda i, eids, vs: (eids[i], 0, 0)),
        ],
        out_specs=pl.BlockSpec((TILE, H), lambda i, eids, vs: (i, 0)),
    )
    expert_ffn = pl.pallas_call(
        _expert_body,
        grid_spec=grid_spec,
        out_shape=jax.ShapeDtypeStruct((PAD, H), f32),
        compiler_params=pltpu.CompilerParams(vmem_limit_bytes=120 * 2**20),
    )
    eo = expert_ffn(tile_eid, tile_valid, xs, w_pad, exp_w_gu, exp_w_down)

    out = _combine(eo, ps_of_a, base)
    return out.reshape(B, S, H)


# speedup vs baseline: 1.8618x; 1.0531x over previous
import functools
import math

import jax
import jax.numpy as jnp
from jax import lax
from jax.experimental import pallas as pl
from jax.experimental.pallas import tpu as pltpu
from jax.experimental.pallas import tpu_sc as plsc

B, S, H = 1, 2048, 768
NH, NKV, HD = 16, 8, 48
E, K, FF = 64, 8, 6144
HHD = HD // 2
G = NH // NKV

TILE = 128
NT = S * K // TILE + E
PAD = NT * TILE

NW = 32
_SC_MESH = dict(core_axis_name="c", subcore_axis_name="s")


def _prologue_body(hid_ref, iw_ref, wq1_ref, wq2_ref, wk1_ref, wk2_ref,
                   wv_ref, cosq_ref, sinq_ref, cosk_ref, sink_ref,
                   qlw1_ref, qlw2_ref, klw1_ref, klw2_ref,
                   sq_ref, sqt_ref, sk_ref, skt_ref,
                   q1_ref, q2_ref, k1_ref, k2_ref, v_ref):
    x = hid_ref[...]
    var = jnp.mean(x * x, axis=1, keepdims=True)
    h = iw_ref[...] * (x * lax.rsqrt(var + 1e-6))
    dot = functools.partial(jnp.dot, preferred_element_type=jnp.float32)
    q1 = dot(h, wq1_ref[...])
    q2 = dot(h, wq2_ref[...])
    k1 = dot(h, wk1_ref[...])
    k2 = dot(h, wk2_ref[...])
    v_ref[...] = dot(h, wv_ref[...])
    cq, sq_ = cosq_ref[...], sinq_ref[...]
    ck, sk_ = cosk_ref[...], sink_ref[...]
    q1r = q1 * cq - q2 * sq_
    q2r = q2 * cq + q1 * sq_
    k1r = k1 * ck - k2 * sk_
    k2r = k2 * ck + k1 * sk_
    ssq = dot(q1r * q1r + q2r * q2r, sq_ref[...])
    scq = lax.rsqrt(ssq / HD + 1e-6)
    sbq = dot(scq, sqt_ref[...])
    ssk = dot(k1r * k1r + k2r * k2r, sk_ref[...])
    sck = lax.rsqrt(ssk / HD + 1e-6)
    sbk = dot(sck, skt_ref[...])
    q1_ref[...] = q1r * sbq * qlw1_ref[...]
    q2_ref[...] = q2r * sbq * qlw2_ref[...]
    k1_ref[...] = k1r * sbk * klw1_ref[...]
    k2_ref[...] = k2r * sbk * klw2_ref[...]


def _attn_body(q_ref, k_ref, v_ref, mask_ref, o_ref):
    q = q_ref[0]
    k = k_ref[0]
    v = v_ref[0]
    s = lax.dot_general(q, k, (((1,), (1,)), ((), ())),
                        preferred_element_type=jnp.float32)
    s = s * (1.0 / math.sqrt(HD)) + mask_ref[...]
    m = jnp.max(s, axis=1, keepdims=True)
    p = jnp.exp(s - m)
    l = jnp.sum(p, axis=1, keepdims=True)
    o = jnp.dot(p, v, preferred_element_type=jnp.float32)
    o_ref[0] = o / l


def _post_body(o2d_ref, hid_ref, ow_ref, pw_ref, gw_ref,
               hs2_ref, h2_ref, logits_ref):
    dot = functools.partial(jnp.dot, preferred_element_type=jnp.float32)
    att = dot(o2d_ref[...], ow_ref[...])
    hs2 = hid_ref[...] + att
    var = jnp.mean(hs2 * hs2, axis=1, keepdims=True)
    h2 = pw_ref[...] * (hs2 * lax.rsqrt(var + 1e-6))
    hs2_ref[...] = hs2
    h2_ref[...] = h2
    logits_ref[...] = dot(h2, gw_ref[...])


def _shared_body(hs2_ref, h2_ref, wgu_ref, wdn_ref, base_ref):
    dot = functools.partial(jnp.dot, preferred_element_type=jnp.float32)
    hm = dot(h2_ref[...], wgu_ref[...])
    x1 = hm[:, :FF // 2]
    x2 = hm[:, FF // 2:]
    act = x1 * (x2 * jax.nn.sigmoid(x2))
    base_ref[...] = hs2_ref[...] + dot(act, wdn_ref[...])


def _expert_body(eids_ref, valid_ref, tok_ref, h2_any, w_ref, wgu_ref,
                 wdn_ref, out_ref, xb, sems):
    i = pl.program_id(0)

    def start_gather(tile_idx, slot):
        for r in range(TILE):
            pltpu.make_async_copy(
                h2_any.at[pl.ds(tok_ref[tile_idx * TILE + r], 1)],
                xb.at[slot, pl.ds(r, 1)],
                sems.at[slot],
            ).start()

    @pl.when(i == 0)
    def _():
        start_gather(0, 0)

    @pl.when(i + 1 < NT)
    def _():
        start_gather(i + 1, (i + 1) % 2)

    slot = i % 2
    pltpu.make_async_copy(h2_any.at[pl.ds(0, TILE)], xb.at[slot],
                          sems.at[slot]).wait()
    out_ref[...] = jnp.zeros_like(out_ref)

    @pl.when(valid_ref[i] == 1)
    def _():
        x = xb[slot]
        hm = jnp.dot(x, wgu_ref[0], preferred_element_type=jnp.float32)
        x1 = hm[:, :FF // 2]
        x2 = hm[:, FF // 2:]
        act = x1 * (x2 * jax.nn.sigmoid(x2))
        o = jnp.dot(act, wdn_ref[0], preferred_element_type=jnp.float32)
        out_ref[...] = o * w_ref[...]


_CTOK = 16


def _combine_body(eo_hbm, psa_hbm, base_hbm, out_hbm, idx_v, rows_v, acc_v, sem):
    wid = lax.axis_index("s") * 2 + lax.axis_index("c")
    tok_per_w = S // NW
    tbase = wid * tok_per_w
    for c in range(tok_per_w // _CTOK):
        t0 = tbase + c * _CTOK
        pltpu.sync_copy(psa_hbm.at[pl.ds(t0 * K, _CTOK * K)], idx_v)
        pltpu.async_copy(eo_hbm.at[idx_v], rows_v, sem).wait()
        pltpu.sync_copy(base_hbm.at[pl.ds(t0, _CTOK)], acc_v)

        def jbody(j, _):
            for t in range(_CTOK):
                a = acc_v[t, pl.ds(j * 16, 16)]
                for k in range(K):
                    a = a + rows_v[t * K + k, pl.ds(j * 16, 16)]
                acc_v[t, pl.ds(j * 16, 16)] = a
            return 0

        lax.fori_loop(0, H // 16, jbody, 0)
        pltpu.sync_copy(acc_v, out_hbm.at[pl.ds(t0, _CTOK)])


def _combine(eo, ps_of_a, base):
    mesh = plsc.VectorSubcoreMesh(**_SC_MESH)
    f = pl.kernel(
        _combine_body,
        out_type=jax.ShapeDtypeStruct((S, H), jnp.float32),
        mesh=mesh,
        scratch_types=[
            pltpu.VMEM((_CTOK * K,), jnp.int32),
            pltpu.VMEM((_CTOK * K, H), jnp.float32),
            pltpu.VMEM((_CTOK, H), jnp.float32),
            pltpu.SemaphoreType.DMA,
        ],
    )
    return f(eo, ps_of_a, base)


def kernel(hidden_states, attention_mask, pos_cos, pos_sin, input_ln_w,
           post_ln_w, qkv_w, o_w, q_ln_w, k_ln_w, gate_w, shared_w_gu,
           shared_w_down, exp_w_gu, exp_w_down):
    f32 = jnp.float32
    x0 = hidden_states.reshape(S, H)

    wr = qkv_w.reshape(H, NKV, G + 2, HD)
    wq = wr[:, :, :G, :].reshape(H, NH, HD)
    wk = wr[:, :, G, :].reshape(H, NKV, HD)
    wv = wr[:, :, G + 1, :].reshape(H, NKV * HD)
    wq1 = wq[:, :, :HHD].reshape(H, NH * HHD)
    wq2 = wq[:, :, HHD:].reshape(H, NH * HHD)
    wk1 = wk[:, :, :HHD].reshape(H, NKV * HHD)
    wk2 = wk[:, :, HHD:].reshape(H, NKV * HHD)
    cos_h = pos_cos[0, :, :HHD]
    sin_h = pos_sin[0, :, :HHD]
    cosq = jnp.tile(cos_h, (1, NH))
    sinq = jnp.tile(sin_h, (1, NH))
    cosk = jnp.tile(cos_h, (1, NKV))
    sink = jnp.tile(sin_h, (1, NKV))
    qlw1 = jnp.tile(q_ln_w[:HHD], NH)[None, :]
    qlw2 = jnp.tile(q_ln_w[HHD:], NH)[None, :]
    klw1 = jnp.tile(k_ln_w[:HHD], NKV)[None, :]
    klw2 = jnp.tile(k_ln_w[HHD:], NKV)[None, :]
    eyeq = jnp.eye(NH, dtype=f32)
    sq_m = jnp.repeat(eyeq, HHD, axis=0)
    sqt_m = jnp.repeat(eyeq, HHD, axis=1)
    eyek = jnp.eye(NKV, dtype=f32)
    sk_m = jnp.repeat(eyek, HHD, axis=0)
    skt_m = jnp.repeat(eyek, HHD, axis=1)

    prologue = pl.pallas_call(
        _prologue_body,
        out_shape=[
            jax.ShapeDtypeStruct((S, NH * HHD), f32),
            jax.ShapeDtypeStruct((S, NH * HHD), f32),
            jax.ShapeDtypeStruct((S, NKV * HHD), f32),
            jax.ShapeDtypeStruct((S, NKV * HHD), f32),
            jax.ShapeDtypeStruct((S, NKV * HD), f32),
        ],
        compiler_params=pltpu.CompilerParams(vmem_limit_bytes=120 * 2**20),
    )
    q1, q2, k1, k2, v2d = prologue(
        x0, input_ln_w[None, :], wq1, wq2, wk1, wk2, wv, cosq, sinq, cosk,
        sink, qlw1, qlw2, klw1, klw2, sq_m, sqt_m, sk_m, skt_m)

    q = jnp.concatenate([q1.reshape(S, NH, HHD), q2.reshape(S, NH, HHD)],
                        axis=-1).transpose(1, 0, 2)
    k = jnp.concatenate([k1.reshape(S, NKV, HHD), k2.reshape(S, NKV, HHD)],
                        axis=-1).transpose(1, 0, 2)
    k = jnp.repeat(k, G, axis=0)
    v = v2d.reshape(S, NKV, HD).transpose(1, 0, 2)
    v = jnp.repeat(v, G, axis=0)
    mask2d = attention_mask.reshape(S, S)

    attn = pl.pallas_call(
        _attn_body,
        grid=(NH,),
        in_specs=[
            pl.BlockSpec((1, S, HD), lambda h: (h, 0, 0)),
            pl.BlockSpec((1, S, HD), lambda h: (h, 0, 0)),
            pl.BlockSpec((1, S, HD), lambda h: (h, 0, 0)),
            pl.BlockSpec((S, S), lambda h: (0, 0)),
        ],
        out_specs=pl.BlockSpec((1, S, HD), lambda h: (h, 0, 0)),
        out_shape=jax.ShapeDtypeStruct((NH, S, HD), f32),
        compiler_params=pltpu.CompilerParams(vmem_limit_bytes=120 * 2**20),
    )
    o3 = attn(q, k, v, mask2d)
    o2d = o3.transpose(1, 0, 2).reshape(S, NH * HD)

    RB = 512
    post = pl.pallas_call(
        _post_body,
        grid=(S // RB,),
        in_specs=[
            pl.BlockSpec((RB, H), lambda i: (i, 0)),
            pl.BlockSpec((RB, H), lambda i: (i, 0)),
            pl.BlockSpec((H, H), lambda i: (0, 0)),
            pl.BlockSpec((1, H), lambda i: (0, 0)),
            pl.BlockSpec((H, E), lambda i: (0, 0)),
        ],
        out_specs=[
            pl.BlockSpec((RB, H), lambda i: (i, 0)),
            pl.BlockSpec((RB, H), lambda i: (i, 0)),
            pl.BlockSpec((RB, E), lambda i: (i, 0)),
        ],
        out_shape=[
            jax.ShapeDtypeStruct((S, H), f32),
            jax.ShapeDtypeStruct((S, H), f32),
            jax.ShapeDtypeStruct((S, E), f32),
        ],
        compiler_params=pltpu.CompilerParams(vmem_limit_bytes=120 * 2**20),
    )
    hs2, h2, logits = post(o2d, x0, o_w, post_ln_w[None, :], gate_w)

    shared = pl.pallas_call(
        _shared_body,
        grid=(S // RB,),
        in_specs=[
            pl.BlockSpec((RB, H), lambda i: (i, 0)),
            pl.BlockSpec((RB, H), lambda i: (i, 0)),
            pl.BlockSpec((H, FF), lambda i: (0, 0)),
            pl.BlockSpec((FF // 2, H), lambda i: (0, 0)),
        ],
        out_specs=pl.BlockSpec((RB, H), lambda i: (i, 0)),
        out_shape=jax.ShapeDtypeStruct((S, H), f32),
        compiler_params=pltpu.CompilerParams(vmem_limit_bytes=120 * 2**20),
    )
    base = shared(hs2, h2, shared_w_gu, shared_w_down)

    i32 = jnp.int32
    gates = jax.nn.softmax(logits, axis=1)
    topv, topi = lax.top_k(gates, K)
    gsum = jnp.maximum(jnp.sum(topv, axis=1, keepdims=True), 1.1920929e-07)
    wts = topv / gsum
    eid = topi.reshape(-1).astype(i32)
    order = jnp.argsort(eid, stable=True).astype(i32)
    counts = jnp.bincount(eid, length=E).astype(i32)
    offs = jnp.concatenate([jnp.zeros((1,), i32),
                            jnp.cumsum(counts)[:-1].astype(i32)])
    nt = (counts + TILE - 1) // TILE
    tcum = jnp.cumsum(nt).astype(i32)
    po = jnp.concatenate([jnp.zeros((1,), i32),
                          tcum[:-1]]) * TILE
    tile_ids = jnp.arange(NT, dtype=i32)
    tile_eid = jnp.searchsorted(tcum, tile_ids, side="right").astype(i32)
    tile_valid = (tile_ids < tcum[-1]).astype(i32)
    tile_eid = jnp.minimum(tile_eid, E - 1)
    ps = jnp.arange(PAD, dtype=i32)
    e_ps = tile_eid[ps // TILE]
    inner = ps - po[e_ps]
    slot_valid = inner < counts[e_ps]
    src = offs[e_ps] + jnp.where(slot_valid, inner, 0)
    a_ps = order[src]
    tok_pad = jnp.where(slot_valid, a_ps // K, 0).astype(i32)
    w_pad = jnp.where(slot_valid, wts.reshape(-1)[a_ps], 0.0)[:, None]
    eid_sorted = eid[order]
    s_idx = jnp.arange(S * K, dtype=i32)
    ps_sorted = po[eid_sorted] + (s_idx - offs[eid_sorted])
    ps_of_a = jnp.zeros((S * K,), i32).at[order].set(ps_sorted)

    grid_spec = pltpu.PrefetchScalarGridSpec(
        num_scalar_prefetch=3,
        grid=(NT,),
        in_specs=[
            pl.BlockSpec(memory_space=pl.ANY),
            pl.BlockSpec((TILE, 1), lambda i, eids, vs, tok: (i, 0)),
            pl.BlockSpec((1, H, FF), lambda i, eids, vs, tok: (eids[i], 0, 0)),
            pl.BlockSpec((1, FF // 2, H),
                         lambda i, eids, vs, tok: (eids[i], 0, 0)),
        ],
        out_specs=pl.BlockSpec((TILE, H), lambda i, eids, vs, tok: (i, 0)),
        scratch_shapes=[
            pltpu.VMEM((2, TILE, H), f32),
            pltpu.SemaphoreType.DMA((2,)),
        ],
    )
    expert_ffn = pl.pallas_call(
        _expert_body,
        grid_spec=grid_spec,
        out_shape=jax.ShapeDtypeStruct((PAD, H), f32),
        compiler_params=pltpu.CompilerParams(vmem_limit_bytes=120 * 2**20),
    )
    eo = expert_ffn(tile_eid, tile_valid, tok_pad, h2, w_pad,
                    exp_w_gu, exp_w_down)

    out = _combine(eo, ps_of_a, base)
    return out.reshape(B, S, H)
